# baseline (speedup 1.0000x reference)
import jax
import jax.numpy as jnp
from jax import lax
from jax.experimental import pallas as pl
from jax.experimental.pallas import tpu as pltpu
from jax.experimental.pallas import tpu_sc as plsc

N = 10000
E = 320000
D = 128

NUM_CORES = 2
NUM_SUBCORES = 16
NUM_WORKERS = NUM_CORES * NUM_SUBCORES
CHUNK = 80
EDGES_PER_WORKER = E // NUM_WORKERS
NCHUNKS = EDGES_PER_WORKER // CHUNK
ROWS_PER_TILE = 624
ROWS_LAST_TILE = N - ROWS_PER_TILE * (NUM_SUBCORES - 1)


def _sc_body(x_hbm, dst_hbm, src_hbm, partials_hbm,
             acc, sidx_all, didx_all, r0, r1, zbuf,
             isem, gs0, gs1):
    rows = [r0, r1]
    gsem = [gs0, gs1]

    c = lax.axis_index("c")
    s = lax.axis_index("s")
    wid = c * NUM_SUBCORES + s
    ebase = pl.multiple_of(wid * EDGES_PER_WORKER, 8)

    def gather(g, t):
        pltpu.async_copy(
            x_hbm.at[sidx_all.at[pl.ds(g * CHUNK, CHUNK)]], rows[t],
            gsem[t])

    def scatter(g, t):
        pltpu.make_async_copy(
            x_hbm.at[pl.ds(0, CHUNK)], rows[t], gsem[t]).wait()
        pltpu.sync_copy(rows[t],
                        acc.at[didx_all.at[pl.ds(g * CHUNK, CHUNK)]],
                        add=True)

    pltpu.async_copy(src_hbm.at[pl.ds(ebase, EDGES_PER_WORKER)], sidx_all,
                     isem)
    pltpu.async_copy(dst_hbm.at[pl.ds(ebase, EDGES_PER_WORKER)], didx_all,
                     isem)

    zero16 = jnp.zeros((16,), jnp.float32)
    for i in range(16):
        for j in range(D // 16):
            zbuf[i, pl.ds(j * 16, 16)] = zero16

    row0 = pl.multiple_of(s * ROWS_PER_TILE, 8)

    @pl.when(s < NUM_SUBCORES - 1)
    def _():
        for r in range(ROWS_PER_TILE // 16):
            pltpu.async_copy(zbuf, acc.at[pl.ds(row0 + r * 16, 16)], gs1)
        for r in range(ROWS_PER_TILE // 16):
            pltpu.make_async_copy(
                zbuf, acc.at[pl.ds(row0 + r * 16, 16)], gs1).wait()

    @pl.when(s == NUM_SUBCORES - 1)
    def _():
        last0 = (NUM_SUBCORES - 1) * ROWS_PER_TILE
        for r in range(ROWS_LAST_TILE // 16):
            pltpu.async_copy(zbuf, acc.at[pl.ds(last0 + r * 16, 16)], gs1)
        for r in range(ROWS_LAST_TILE // 16):
            pltpu.make_async_copy(
                zbuf, acc.at[pl.ds(last0 + r * 16, 16)], gs1).wait()

    pltpu.make_async_copy(
        src_hbm.at[pl.ds(0, EDGES_PER_WORKER)], sidx_all, isem).wait()
    pltpu.make_async_copy(
        dst_hbm.at[pl.ds(0, EDGES_PER_WORKER)], didx_all, isem).wait()

    plsc.subcore_barrier()

    gather(0, 0)

    def body(k, carry):
        g = 2 * k
        gather(g + 1, 1)
        scatter(g, 0)

        @pl.when(g + 2 < NCHUNKS)
        def _():
            gather(g + 2, 0)

        scatter(g + 1, 1)
        return carry

    lax.fori_loop(0, NCHUNKS // 2, body, 0)

    scatter(NCHUNKS - 1, 0)

    plsc.subcore_barrier()

    @pl.when(s < NUM_SUBCORES - 1)
    def _():
        pltpu.sync_copy(acc.at[pl.ds(row0, ROWS_PER_TILE)],
                        partials_hbm.at[c, pl.ds(row0, ROWS_PER_TILE)])

    @pl.when(s == NUM_SUBCORES - 1)
    def _():
        last0 = (NUM_SUBCORES - 1) * ROWS_PER_TILE
        pltpu.sync_copy(acc.at[pl.ds(last0, ROWS_LAST_TILE)],
                        partials_hbm.at[c, pl.ds(last0, ROWS_LAST_TILE)])


@jax.jit
def _sc_aggregate(x, dst, src):
    mesh = plsc.VectorSubcoreMesh(core_axis_name="c", subcore_axis_name="s")
    k = pl.kernel(
        _sc_body,
        out_type=jax.ShapeDtypeStruct((NUM_CORES, N, D), jnp.float32),
        mesh=mesh,
        scratch_types=(
            [pltpu.VMEM_SHARED((N, D), jnp.float32)]
            + [pltpu.VMEM((EDGES_PER_WORKER,), jnp.int32) for _ in range(2)]
            + [pltpu.VMEM((CHUNK, D), jnp.float32) for _ in range(2)]
            + [pltpu.VMEM((16, D), jnp.float32)]
            + [pltpu.SemaphoreType.DMA for _ in range(3)]
        ),
    )
    return k(x, dst, src)


def _tc_body(p_ref, w_ref, o_ref):
    summed = p_ref[0] + p_ref[1]
    o_ref[...] = jnp.maximum(
        jnp.dot(summed, w_ref[...], preferred_element_type=jnp.float32), 0.0)


@jax.jit
def _tc_matmul_relu(partials, W):
    BLOCK = 1000
    return pl.pallas_call(
        _tc_body,
        out_shape=jax.ShapeDtypeStruct((N, D), jnp.float32),
        grid=(N // BLOCK,),
        in_specs=[
            pl.BlockSpec((NUM_CORES, BLOCK, D), lambda i: (0, i, 0)),
            pl.BlockSpec((D, D), lambda i: (0, 0)),
        ],
        out_specs=pl.BlockSpec((BLOCK, D), lambda i: (i, 0)),
    )(partials, W)


def kernel(x, edge_index, W):
    dst = edge_index[0]
    src = edge_index[1]
    partials = _sc_aggregate(x, dst, src)
    return _tc_matmul_relu(partials, W)


# speedup vs baseline: 1.0018x; 1.0018x over previous
import jax
import jax.numpy as jnp
from jax import lax
from jax.experimental import pallas as pl
from jax.experimental.pallas import tpu as pltpu
from jax.experimental.pallas import tpu_sc as plsc

N = 10000
E = 320000
D = 128

NUM_CORES = 2
NUM_SUBCORES = 16
NUM_WORKERS = NUM_CORES * NUM_SUBCORES
CHUNK = 80
EDGES_PER_WORKER = E // NUM_WORKERS
NCHUNKS = EDGES_PER_WORKER // CHUNK
ROWS_PER_TILE = 624
ROWS_LAST_TILE = N - ROWS_PER_TILE * (NUM_SUBCORES - 1)


def _sc_body(x_hbm, dst_hbm, src_hbm, partials_hbm,
             acc, sidx_all, didx_all, r0, r1, zbuf,
             isem, gs0, gs1):
    rows = [r0, r1]
    gsem = [gs0, gs1]

    c = lax.axis_index("c")
    s = lax.axis_index("s")
    wid = c * NUM_SUBCORES + s
    ebase = pl.multiple_of(wid * EDGES_PER_WORKER, 8)

    H = CHUNK // 2

    def gather(g, t):
        pltpu.async_copy(
            x_hbm.at[sidx_all.at[pl.ds(g * CHUNK, H)]],
            rows[t].at[pl.ds(0, H)], gsem[t])
        pltpu.async_copy(
            x_hbm.at[sidx_all.at[pl.ds(g * CHUNK + H, H)]],
            rows[t].at[pl.ds(H, H)], gsem[t])

    def scatter(g, t):
        pltpu.make_async_copy(
            x_hbm.at[pl.ds(0, CHUNK)], rows[t], gsem[t]).wait()
        pltpu.sync_copy(rows[t],
                        acc.at[didx_all.at[pl.ds(g * CHUNK, CHUNK)]],
                        add=True)

    pltpu.async_copy(src_hbm.at[pl.ds(ebase, EDGES_PER_WORKER)], sidx_all,
                     isem)
    pltpu.async_copy(dst_hbm.at[pl.ds(ebase, EDGES_PER_WORKER)], didx_all,
                     isem)

    zero16 = jnp.zeros((16,), jnp.float32)
    for i in range(16):
        for j in range(D // 16):
            zbuf[i, pl.ds(j * 16, 16)] = zero16

    row0 = pl.multiple_of(s * ROWS_PER_TILE, 8)

    @pl.when(s < NUM_SUBCORES - 1)
    def _():
        for r in range(ROWS_PER_TILE // 16):
            pltpu.async_copy(zbuf, acc.at[pl.ds(row0 + r * 16, 16)], gs1)
        for r in range(ROWS_PER_TILE // 16):
            pltpu.make_async_copy(
                zbuf, acc.at[pl.ds(row0 + r * 16, 16)], gs1).wait()

    @pl.when(s == NUM_SUBCORES - 1)
    def _():
        last0 = (NUM_SUBCORES - 1) * ROWS_PER_TILE
        for r in range(ROWS_LAST_TILE // 16):
            pltpu.async_copy(zbuf, acc.at[pl.ds(last0 + r * 16, 16)], gs1)
        for r in range(ROWS_LAST_TILE // 16):
            pltpu.make_async_copy(
                zbuf, acc.at[pl.ds(last0 + r * 16, 16)], gs1).wait()

    pltpu.make_async_copy(
        src_hbm.at[pl.ds(0, EDGES_PER_WORKER)], sidx_all, isem).wait()
    pltpu.make_async_copy(
        dst_hbm.at[pl.ds(0, EDGES_PER_WORKER)], didx_all, isem).wait()

    plsc.subcore_barrier()

    gather(0, 0)

    def body(k, carry):
        g = 2 * k
        gather(g + 1, 1)
        scatter(g, 0)

        @pl.when(g + 2 < NCHUNKS)
        def _():
            gather(g + 2, 0)

        scatter(g + 1, 1)
        return carry

    lax.fori_loop(0, NCHUNKS // 2, body, 0)

    scatter(NCHUNKS - 1, 0)

    plsc.subcore_barrier()

    @pl.when(s < NUM_SUBCORES - 1)
    def _():
        pltpu.sync_copy(acc.at[pl.ds(row0, ROWS_PER_TILE)],
                        partials_hbm.at[c, pl.ds(row0, ROWS_PER_TILE)])

    @pl.when(s == NUM_SUBCORES - 1)
    def _():
        last0 = (NUM_SUBCORES - 1) * ROWS_PER_TILE
        pltpu.sync_copy(acc.at[pl.ds(last0, ROWS_LAST_TILE)],
                        partials_hbm.at[c, pl.ds(last0, ROWS_LAST_TILE)])


@jax.jit
def _sc_aggregate(x, dst, src):
    mesh = plsc.VectorSubcoreMesh(core_axis_name="c", subcore_axis_name="s")
    k = pl.kernel(
        _sc_body,
        out_type=jax.ShapeDtypeStruct((NUM_CORES, N, D), jnp.float32),
        mesh=mesh,
        scratch_types=(
            [pltpu.VMEM_SHARED((N, D), jnp.float32)]
            + [pltpu.VMEM((EDGES_PER_WORKER,), jnp.int32) for _ in range(2)]
            + [pltpu.VMEM((CHUNK, D), jnp.float32) for _ in range(2)]
            + [pltpu.VMEM((16, D), jnp.float32)]
            + [pltpu.SemaphoreType.DMA for _ in range(3)]
        ),
    )
    return k(x, dst, src)


def _tc_body(p_ref, w_ref, o_ref):
    summed = p_ref[0] + p_ref[1]
    o_ref[...] = jnp.maximum(
        jnp.dot(summed, w_ref[...], preferred_element_type=jnp.float32), 0.0)


@jax.jit
def _tc_matmul_relu(partials, W):
    BLOCK = 1000
    return pl.pallas_call(
        _tc_body,
        out_shape=jax.ShapeDtypeStruct((N, D), jnp.float32),
        grid=(N // BLOCK,),
        in_specs=[
            pl.BlockSpec((NUM_CORES, BLOCK, D), lambda i: (0, i, 0)),
            pl.BlockSpec((D, D), lambda i: (0, 0)),
        ],
        out_specs=pl.BlockSpec((BLOCK, D), lambda i: (i, 0)),
    )(partials, W)


def kernel(x, edge_index, W):
    dst = edge_index[0]
    src = edge_index[1]
    partials = _sc_aggregate(x, dst, src)
    return _tc_matmul_relu(partials, W)


# speedup vs baseline: 1.0055x; 1.0037x over previous
import jax
import jax.numpy as jnp
from jax import lax
from jax.experimental import pallas as pl
from jax.experimental.pallas import tpu as pltpu
from jax.experimental.pallas import tpu_sc as plsc

N = 10000
E = 320000
D = 128

NUM_CORES = 2
NUM_SUBCORES = 16
NUM_WORKERS = NUM_CORES * NUM_SUBCORES
CHUNK = 80
EDGES_PER_WORKER = E // NUM_WORKERS
NCHUNKS = EDGES_PER_WORKER // CHUNK
ROWS_PER_TILE = 624
ROWS_LAST_TILE = N - ROWS_PER_TILE * (NUM_SUBCORES - 1)


def _sc_body(x_hbm, dst_hbm, src_hbm, partials_hbm,
             acc, s0, s1, s2, d0, d1, d2, r0, r1, r2, zbuf,
             is0, is1, is2, gs0, gs1, gs2):
    sidx = [s0, s1, s2]
    didx = [d0, d1, d2]
    rows = [r0, r1, r2]
    isem = [is0, is1, is2]
    gsem = [gs0, gs1, gs2]

    c = lax.axis_index("c")
    s = lax.axis_index("s")
    wid = c * NUM_SUBCORES + s
    ebase = wid * EDGES_PER_WORKER

    def idx_load(g, t):
        base = pl.multiple_of(ebase + g * CHUNK, 8)
        pltpu.async_copy(src_hbm.at[pl.ds(base, CHUNK)], sidx[t], isem[t])
        pltpu.async_copy(dst_hbm.at[pl.ds(base, CHUNK)], didx[t], isem[t])

    def gather(t):
        pltpu.make_async_copy(
            src_hbm.at[pl.ds(0, CHUNK)], sidx[t], isem[t]).wait()
        pltpu.make_async_copy(
            dst_hbm.at[pl.ds(0, CHUNK)], didx[t], isem[t]).wait()
        pltpu.async_copy(x_hbm.at[sidx[t]], rows[t], gsem[t])

    def scatter(t):
        pltpu.make_async_copy(
            x_hbm.at[pl.ds(0, CHUNK)], rows[t], gsem[t]).wait()
        pltpu.sync_copy(rows[t], acc.at[didx[t]], add=True)

    idx_load(0, 0)
    idx_load(1, 1)
    idx_load(2, 2)
    gather(0)

    zero16 = jnp.zeros((16,), jnp.float32)
    for i in range(16):
        for j in range(D // 16):
            zbuf[i, pl.ds(j * 16, 16)] = zero16

    row0 = pl.multiple_of(s * ROWS_PER_TILE, 8)

    @pl.when(s < NUM_SUBCORES - 1)
    def _():
        for r in range(ROWS_PER_TILE // 16):
            pltpu.async_copy(zbuf, acc.at[pl.ds(row0 + r * 16, 16)], gs1)
        for r in range(ROWS_PER_TILE // 16):
            pltpu.make_async_copy(
                zbuf, acc.at[pl.ds(row0 + r * 16, 16)], gs1).wait()

    @pl.when(s == NUM_SUBCORES - 1)
    def _():
        last0 = (NUM_SUBCORES - 1) * ROWS_PER_TILE
        for r in range(ROWS_LAST_TILE // 16):
            pltpu.async_copy(zbuf, acc.at[pl.ds(last0 + r * 16, 16)], gs1)
        for r in range(ROWS_LAST_TILE // 16):
            pltpu.make_async_copy(
                zbuf, acc.at[pl.ds(last0 + r * 16, 16)], gs1).wait()

    plsc.subcore_barrier()

    def body(k, carry):
        g = 3 * k
        gather(1)
        scatter(0)
        idx_load(g + 3, 0)
        gather(2)
        scatter(1)
        idx_load(g + 4, 1)
        gather(0)
        scatter(2)

        @pl.when(g + 5 < NCHUNKS)
        def _():
            idx_load(g + 5, 2)

        return carry

    lax.fori_loop(0, (NCHUNKS - 2) // 3, body, 0)

    gather(1)
    scatter(0)
    scatter(1)

    plsc.subcore_barrier()

    @pl.when(s < NUM_SUBCORES - 1)
    def _():
        pltpu.sync_copy(acc.at[pl.ds(row0, ROWS_PER_TILE)],
                        partials_hbm.at[c, pl.ds(row0, ROWS_PER_TILE)])

    @pl.when(s == NUM_SUBCORES - 1)
    def _():
        last0 = (NUM_SUBCORES - 1) * ROWS_PER_TILE
        pltpu.sync_copy(acc.at[pl.ds(last0, ROWS_LAST_TILE)],
                        partials_hbm.at[c, pl.ds(last0, ROWS_LAST_TILE)])


@jax.jit
def _sc_aggregate(x, dst, src):
    mesh = plsc.VectorSubcoreMesh(core_axis_name="c", subcore_axis_name="s")
    k = pl.kernel(
        _sc_body,
        out_type=jax.ShapeDtypeStruct((NUM_CORES, N, D), jnp.float32),
        mesh=mesh,
        scratch_types=(
            [pltpu.VMEM_SHARED((N, D), jnp.float32)]
            + [pltpu.VMEM((CHUNK,), jnp.int32) for _ in range(6)]
            + [pltpu.VMEM((CHUNK, D), jnp.float32) for _ in range(3)]
            + [pltpu.VMEM((16, D), jnp.float32)]
            + [pltpu.SemaphoreType.DMA for _ in range(6)]
        ),
    )
    return k(x, dst, src)


def _tc_body(p_ref, w_ref, o_ref):
    summed = p_ref[0] + p_ref[1]
    o_ref[...] = jnp.maximum(
        jnp.dot(summed, w_ref[...], preferred_element_type=jnp.float32), 0.0)


@jax.jit
def _tc_matmul_relu(partials, W):
    BLOCK = 1000
    return pl.pallas_call(
        _tc_body,
        out_shape=jax.ShapeDtypeStruct((N, D), jnp.float32),
        grid=(N // BLOCK,),
        in_specs=[
            pl.BlockSpec((NUM_CORES, BLOCK, D), lambda i: (0, i, 0)),
            pl.BlockSpec((D, D), lambda i: (0, 0)),
        ],
        out_specs=pl.BlockSpec((BLOCK, D), lambda i: (i, 0)),
    )(partials, W)


def kernel(x, edge_index, W):
    dst = edge_index[0]
    src = edge_index[1]
    partials = _sc_aggregate(x, dst, src)
    return _tc_matmul_relu(partials, W)
